# Initial kernel scaffold; baseline (speedup 1.0000x reference)
import jax
import jax.numpy as jnp
from jax.experimental import pallas as pl


def kernel(x, edge_index, batch, Wl1, bl1, Wr1, Wl2, bl2, Wr2, Wl3, bl3, Wr3):
    raise NotImplementedError("write your pallas kernel here")


# speedup vs baseline: 4.6826x; 4.6826x over previous
import functools

import jax
import jax.numpy as jnp
from jax import lax
from jax.experimental import pallas as pl
from jax.experimental.pallas import tpu as pltpu
from jax.experimental.pallas import tpu_sc as plsc

K = 30
B = 64
N = 10000
E = 160000
H = 256

NC = 2
NS = 16
CH = H // NC
EPT = E // NS
C = 125
NCHUNK = EPT // C
NPAD = 10240
STRIPE = NPAD // NS
NP2 = 10240
RB = 1000


@functools.cache
def _make_agg():
  mesh = plsc.VectorSubcoreMesh(
      core_axis_name="c", subcore_axis_name="s", num_cores=NC, num_subcores=NS)

  def body(hsplit, srcidx, dstidx, zr, aggout, srcv, dstv, rowsv, acc):
    c = lax.axis_index("c")
    s = lax.axis_index("s")
    pltpu.sync_copy(zr, acc.at[pl.ds(s * STRIPE, STRIPE)])
    pltpu.sync_copy(srcidx.at[c, s], srcv)
    pltpu.sync_copy(dstidx.at[s], dstv)
    plsc.subcore_barrier()
    def chunk(j, carry):
      pltpu.sync_copy(hsplit.at[srcv.at[j]], rowsv)
      pltpu.sync_copy(rowsv, acc.at[dstv.at[j]], add=True)
      return carry
    lax.fori_loop(0, NCHUNK, chunk, 0)
    plsc.subcore_barrier()
    pltpu.sync_copy(acc.at[pl.ds(s * STRIPE, STRIPE)],
                    aggout.at[c, pl.ds(s * STRIPE, STRIPE)])

  return pl.kernel(
      body,
      out_type=jax.ShapeDtypeStruct((NC, NPAD, CH), jnp.float32),
      mesh=mesh,
      scratch_types=[
          pltpu.VMEM((NCHUNK, C), jnp.int32),
          pltpu.VMEM((NCHUNK, C), jnp.int32),
          pltpu.VMEM((C, CH), jnp.float32),
          pltpu.VMEM_SHARED((NPAD, CH), jnp.float32),
      ])


@functools.cache
def _make_cnt():
  mesh = plsc.VectorSubcoreMesh(
      core_axis_name="c", subcore_axis_name="s", num_cores=NC, num_subcores=NS)

  def body(dstidx, zr, ones_h, cntout, dstv, onesv, cacc):
    c = lax.axis_index("c")
    s = lax.axis_index("s")
    @pl.when(c == 0)
    def _():
      pltpu.sync_copy(zr, cacc.at[pl.ds(s * STRIPE, STRIPE)])
      pltpu.sync_copy(dstidx.at[s], dstv)
      pltpu.sync_copy(ones_h, onesv)
    plsc.subcore_barrier()
    @pl.when(c == 0)
    def _():
      def chunk(j, carry):
        pltpu.sync_copy(onesv, cacc.at[dstv.at[j]], add=True)
        return carry
      lax.fori_loop(0, NCHUNK, chunk, 0)
    plsc.subcore_barrier()
    @pl.when(c == 0)
    def _():
      pltpu.sync_copy(cacc.at[pl.ds(s * STRIPE, STRIPE)],
                      cntout.at[pl.ds(s * STRIPE, STRIPE)])

  return pl.kernel(
      body,
      out_type=jax.ShapeDtypeStruct((NPAD, CH), jnp.float32),
      mesh=mesh,
      scratch_types=[
          pltpu.VMEM((NCHUNK, C), jnp.int32),
          pltpu.VMEM((C, CH), jnp.float32),
          pltpu.VMEM_SHARED((NPAD, CH), jnp.float32),
      ])


def _mm_body(agg_ref, cnt_ref, h_ref, wl_ref, bl_ref, wr_ref, o_ref):
  inv = 1.0 / jnp.maximum(cnt_ref[...], 1.0)
  a = agg_ref[...] * inv
  o = (jnp.dot(a, wl_ref[...], preferred_element_type=jnp.float32,
               precision=lax.Precision.HIGHEST)
       + bl_ref[...]
       + jnp.dot(h_ref[...], wr_ref[...], preferred_element_type=jnp.float32,
                 precision=lax.Precision.HIGHEST))
  o_ref[...] = jnp.maximum(o, 0.0)


def _sage_dense(agg, cnt2, h, wl, bl2, wr):
  return pl.pallas_call(
      _mm_body,
      grid=(N // RB,),
      in_specs=[
          pl.BlockSpec((RB, H), lambda i: (i, 0)),
          pl.BlockSpec((RB, 1), lambda i: (i, 0)),
          pl.BlockSpec((RB, H), lambda i: (i, 0)),
          pl.BlockSpec((H, H), lambda i: (0, 0)),
          pl.BlockSpec((1, H), lambda i: (0, 0)),
          pl.BlockSpec((H, H), lambda i: (0, 0)),
      ],
      out_specs=pl.BlockSpec((RB, H), lambda i: (i, 0)),
      out_shape=jax.ShapeDtypeStruct((N, H), jnp.float32),
  )(agg, cnt2, h, wl, bl2, wr)


def _pool_body(keys_ref, bt_ref, h_ref, o_ref):
  g = pl.program_id(0)
  neg = jnp.float32(-jnp.inf)
  rowid = lax.broadcasted_iota(jnp.int32, (NP2 // 128, 128), 0)
  colid = lax.broadcasted_iota(jnp.int32, (NP2 // 128, 128), 1)
  flat = rowid * 128 + colid
  big = jnp.int32(2 ** 30)
  masked = jnp.where(bt_ref[...] == g, keys_ref[...], neg)
  for k in range(K):
    m = jnp.max(masked)
    idx = jnp.min(jnp.where(masked == m, flat, big))
    valid = m > neg
    idxc = jnp.minimum(idx, N - 1)
    row = h_ref[pl.ds(idxc, 1), :]
    o_ref[pl.ds(g, 1), k * H:(k + 1) * H] = jnp.where(valid, row, 0.0)
    masked = jnp.where(flat == idx, neg, masked)


def _sort_pool(keys, bt, h):
  return pl.pallas_call(
      _pool_body,
      grid=(B,),
      in_specs=[
          pl.BlockSpec((NP2 // 128, 128), lambda g: (0, 0)),
          pl.BlockSpec((NP2 // 128, 128), lambda g: (0, 0)),
          pl.BlockSpec((N, H), lambda g: (0, 0)),
      ],
      out_specs=pl.BlockSpec((B, K * H), lambda g: (0, 0)),
      out_shape=jax.ShapeDtypeStruct((B, K * H), jnp.float32),
  )(keys, bt, h)


def _split(h):
  return jnp.concatenate([h[:, :CH], h[:, CH:]], axis=0)


def _unsplit(aggout):
  return jnp.concatenate([aggout[0, :N], aggout[1, :N]], axis=1)


@jax.jit
def kernel(x, edge_index, batch, Wl1, bl1, Wr1, Wl2, bl2, Wr2, Wl3, bl3, Wr3):
  src = edge_index[0].astype(jnp.int32)
  dst = edge_index[1].astype(jnp.int32)
  src_t = src.reshape(NS, NCHUNK, C)
  srcidx = jnp.stack([src_t, src_t + N])
  dstidx = dst.reshape(NS, NCHUNK, C)
  zr = jnp.zeros((STRIPE, CH), jnp.float32)
  ones_h = jnp.ones((C, CH), jnp.float32)

  cntout = _make_cnt()(dstidx, zr, ones_h)
  cnt2 = cntout[:N, :1]

  h = x
  for wl, bl, wr in ((Wl1, bl1, Wr1), (Wl2, bl2, Wr2), (Wl3, bl3, Wr3)):
    hs = _split(h)
    aggout = _make_agg()(hs, srcidx, dstidx, zr)
    agg = _unsplit(aggout)
    h = _sage_dense(agg, cnt2, h, wl, bl.reshape(1, H), wr)

  keys = jnp.concatenate(
      [h[:, -1], jnp.full((NP2 - N,), -jnp.inf, jnp.float32)]
  ).reshape(NP2 // 128, 128)
  bt = jnp.concatenate(
      [batch.astype(jnp.int32), jnp.full((NP2 - N,), -1, jnp.int32)]
  ).reshape(NP2 // 128, 128)
  return _sort_pool(keys, bt, h)


# speedup vs baseline: 7.3729x; 1.5745x over previous
import functools

import jax
import jax.numpy as jnp
from jax import lax
from jax.experimental import pallas as pl
from jax.experimental.pallas import tpu as pltpu
from jax.experimental.pallas import tpu_sc as plsc

K = 30
B = 64
N = 10000
E = 160000
H = 256

NC = 2
NS = 16
CH = H // NC
EPT = E // NS
C = 125
NCHUNK = EPT // C
NPAD = 10240
STRIPE = NPAD // NS
NP2 = 10240
RB = 1000


@functools.cache
def _make_agg():
  mesh = plsc.VectorSubcoreMesh(
      core_axis_name="c", subcore_axis_name="s", num_cores=NC, num_subcores=NS)

  def body(hsplit, srcidx, dstidx, zr, aggout, srcv, dstv, rowsv, acc):
    c = lax.axis_index("c")
    s = lax.axis_index("s")
    pltpu.sync_copy(zr, acc.at[pl.ds(s * STRIPE, STRIPE)])
    pltpu.sync_copy(srcidx.at[c, s], srcv)
    pltpu.sync_copy(dstidx.at[s], dstv)
    plsc.subcore_barrier()
    def chunk(j, carry):
      pltpu.sync_copy(hsplit.at[srcv.at[j]], rowsv)
      pltpu.sync_copy(rowsv, acc.at[dstv.at[j]], add=True)
      return carry
    lax.fori_loop(0, NCHUNK, chunk, 0)
    plsc.subcore_barrier()
    pltpu.sync_copy(acc.at[pl.ds(s * STRIPE, STRIPE)],
                    aggout.at[c, pl.ds(s * STRIPE, STRIPE)])

  return pl.kernel(
      body,
      out_type=jax.ShapeDtypeStruct((NC, NPAD, CH), jnp.float32),
      mesh=mesh,
      scratch_types=[
          pltpu.VMEM((NCHUNK, C), jnp.int32),
          pltpu.VMEM((NCHUNK, C), jnp.int32),
          pltpu.VMEM((C, CH), jnp.float32),
          pltpu.VMEM_SHARED((NPAD, CH), jnp.float32),
      ])


@functools.cache
def _make_cnt():
  mesh = plsc.VectorSubcoreMesh(
      core_axis_name="c", subcore_axis_name="s", num_cores=NC, num_subcores=NS)

  def body(dstidx, zr, ones_h, cntout, dstv, onesv, cacc):
    c = lax.axis_index("c")
    s = lax.axis_index("s")
    @pl.when(c == 0)
    def _():
      pltpu.sync_copy(zr, cacc.at[pl.ds(s * STRIPE, STRIPE)])
      pltpu.sync_copy(dstidx.at[s], dstv)
      pltpu.sync_copy(ones_h, onesv)
    plsc.subcore_barrier()
    @pl.when(c == 0)
    def _():
      def chunk(j, carry):
        pltpu.sync_copy(onesv, cacc.at[dstv.at[j]], add=True)
        return carry
      lax.fori_loop(0, NCHUNK, chunk, 0)
    plsc.subcore_barrier()
    @pl.when(c == 0)
    def _():
      pltpu.sync_copy(cacc.at[pl.ds(s * STRIPE, STRIPE)],
                      cntout.at[pl.ds(s * STRIPE, STRIPE)])

  return pl.kernel(
      body,
      out_type=jax.ShapeDtypeStruct((NPAD, CH), jnp.float32),
      mesh=mesh,
      scratch_types=[
          pltpu.VMEM((NCHUNK, C), jnp.int32),
          pltpu.VMEM((C, CH), jnp.float32),
          pltpu.VMEM_SHARED((NPAD, CH), jnp.float32),
      ])


def _mm_body(agg_ref, cnt_ref, h_ref, wl_ref, bl_ref, wr_ref, o_ref):
  inv = 1.0 / jnp.maximum(cnt_ref[...], 1.0)
  a = agg_ref[...] * inv
  o = (jnp.dot(a, wl_ref[...], preferred_element_type=jnp.float32,
               precision=lax.Precision.HIGHEST)
       + bl_ref[...]
       + jnp.dot(h_ref[...], wr_ref[...], preferred_element_type=jnp.float32,
                 precision=lax.Precision.HIGHEST))
  o_ref[...] = jnp.maximum(o, 0.0)


def _sage_dense(agg, cnt2, h, wl, bl2, wr):
  return pl.pallas_call(
      _mm_body,
      grid=(N // RB,),
      in_specs=[
          pl.BlockSpec((RB, H), lambda i: (i, 0)),
          pl.BlockSpec((RB, 1), lambda i: (i, 0)),
          pl.BlockSpec((RB, H), lambda i: (i, 0)),
          pl.BlockSpec((H, H), lambda i: (0, 0)),
          pl.BlockSpec((1, H), lambda i: (0, 0)),
          pl.BlockSpec((H, H), lambda i: (0, 0)),
      ],
      out_specs=pl.BlockSpec((RB, H), lambda i: (i, 0)),
      out_shape=jax.ShapeDtypeStruct((N, H), jnp.float32),
  )(agg, cnt2, h, wl, bl2, wr)


GB = 8


def _pool_idx_body(keys_ref, bt_ref, oi_ref):
  g0 = pl.program_id(0) * GB
  neg = jnp.float32(-jnp.inf)
  big = jnp.int32(2 ** 30)
  gvec = g0 + lax.broadcasted_iota(jnp.int32, (GB, 1), 0)
  flat = lax.broadcasted_iota(jnp.int32, (GB, NP2), 1)
  masked = jnp.where(bt_ref[...] == gvec, keys_ref[...], neg)
  for k in range(K):
    m = jnp.max(masked, axis=1, keepdims=True)
    idx = jnp.min(jnp.where(masked == m, flat, big), axis=1, keepdims=True)
    oi_ref[:, k:k + 1] = jnp.where(m > neg, idx, N)
    masked = jnp.where(flat == idx, neg, masked)


def _pool_idx(keys, bt):
  return pl.pallas_call(
      _pool_idx_body,
      grid=(B // GB,),
      in_specs=[
          pl.BlockSpec((1, NP2), lambda i: (0, 0)),
          pl.BlockSpec((1, NP2), lambda i: (0, 0)),
      ],
      out_specs=pl.BlockSpec((GB, K), lambda i: (i, 0)),
      out_shape=jax.ShapeDtypeStruct((B, K), jnp.int32),
  )(keys, bt)


GROWS = B * K // (NC * NS)
GPAD = 64


@functools.cache
def _make_rowgather():
  mesh = plsc.VectorSubcoreMesh(
      core_axis_name="c", subcore_axis_name="s", num_cores=NC, num_subcores=NS)

  def body(hpad, idx, out, idxv, rowsv):
    c = lax.axis_index("c")
    s = lax.axis_index("s")
    w = c * NS + s
    pltpu.sync_copy(idx.at[w], idxv)
    pltpu.sync_copy(hpad.at[idxv], rowsv)
    pltpu.sync_copy(rowsv, out.at[w])

  return pl.kernel(
      body,
      out_type=jax.ShapeDtypeStruct((NC * NS, GPAD, H), jnp.float32),
      mesh=mesh,
      scratch_types=[
          pltpu.VMEM((GPAD,), jnp.int32),
          pltpu.VMEM((GPAD, H), jnp.float32),
      ])


def _split(h):
  return jnp.concatenate([h[:, :CH], h[:, CH:]], axis=0)


def _unsplit(aggout):
  return jnp.concatenate([aggout[0, :N], aggout[1, :N]], axis=1)


@jax.jit
def kernel(x, edge_index, batch, Wl1, bl1, Wr1, Wl2, bl2, Wr2, Wl3, bl3, Wr3):
  src = edge_index[0].astype(jnp.int32)
  dst = edge_index[1].astype(jnp.int32)
  src_t = src.reshape(NS, NCHUNK, C)
  srcidx = jnp.stack([src_t, src_t + N])
  dstidx = dst.reshape(NS, NCHUNK, C)
  zr = jnp.zeros((STRIPE, CH), jnp.float32)
  ones_h = jnp.ones((C, CH), jnp.float32)

  cntout = _make_cnt()(dstidx, zr, ones_h)
  cnt2 = cntout[:N, :1]

  h = x
  for wl, bl, wr in ((Wl1, bl1, Wr1), (Wl2, bl2, Wr2), (Wl3, bl3, Wr3)):
    hs = _split(h)
    aggout = _make_agg()(hs, srcidx, dstidx, zr)
    agg = _unsplit(aggout)
    h = _sage_dense(agg, cnt2, h, wl, bl.reshape(1, H), wr)

  keys = jnp.concatenate(
      [h[:, -1], jnp.full((NP2 - N,), -jnp.inf, jnp.float32)]
  ).reshape(1, NP2)
  bt = jnp.concatenate(
      [batch.astype(jnp.int32), jnp.full((NP2 - N,), -1, jnp.int32)]
  ).reshape(1, NP2)
  topidx = _pool_idx(keys, bt)
  hpad = jnp.concatenate([h, jnp.zeros((16, H), jnp.float32)], axis=0)
  idx64 = jnp.full((NC * NS, GPAD), N, jnp.int32)
  idx64 = idx64.at[:, :GROWS].set(topidx.reshape(NC * NS, GROWS))
  rows = _make_rowgather()(hpad, idx64)
  return rows[:, :GROWS].reshape(B, K * H)


# speedup vs baseline: 7.5742x; 1.0273x over previous
import functools

import jax
import jax.numpy as jnp
from jax import lax
from jax.experimental import pallas as pl
from jax.experimental.pallas import tpu as pltpu
from jax.experimental.pallas import tpu_sc as plsc

K = 30
B = 64
N = 10000
E = 160000
H = 256

NC = 2
NS = 16
CH = H // NC
EPT = E // NS
C = 125
NCHUNK = EPT // C
NPAD = 10240
STRIPE = NPAD // NS
NP2 = 10240
RB = 1000


@functools.cache
def _make_agg():
  mesh = plsc.VectorSubcoreMesh(
      core_axis_name="c", subcore_axis_name="s", num_cores=NC, num_subcores=NS)

  def body(hsplit, srcidx, dstidx, zr, aggout, srcv, dstv, rowsv, acc):
    c = lax.axis_index("c")
    s = lax.axis_index("s")
    pltpu.sync_copy(zr, acc.at[pl.ds(s * STRIPE, STRIPE)])
    pltpu.sync_copy(srcidx.at[c, s], srcv)
    pltpu.sync_copy(dstidx.at[s], dstv)
    plsc.subcore_barrier()
    def chunk(j, carry):
      pltpu.sync_copy(hsplit.at[srcv.at[j]], rowsv)
      pltpu.sync_copy(rowsv, acc.at[dstv.at[j]], add=True)
      return carry
    lax.fori_loop(0, NCHUNK, chunk, 0)
    plsc.subcore_barrier()
    pltpu.sync_copy(acc.at[pl.ds(s * STRIPE, STRIPE)],
                    aggout.at[c, pl.ds(s * STRIPE, STRIPE)])

  return pl.kernel(
      body,
      out_type=jax.ShapeDtypeStruct((NC, NPAD, CH), jnp.float32),
      mesh=mesh,
      scratch_types=[
          pltpu.VMEM((NCHUNK, C), jnp.int32),
          pltpu.VMEM((NCHUNK, C), jnp.int32),
          pltpu.VMEM((C, CH), jnp.float32),
          pltpu.VMEM_SHARED((NPAD, CH), jnp.float32),
      ])


@functools.cache
def _make_cnt():
  mesh = plsc.VectorSubcoreMesh(
      core_axis_name="c", subcore_axis_name="s", num_cores=NC, num_subcores=NS)

  def body(dstidx, zr, ones_h, cntout, dstv, onesv, cacc):
    c = lax.axis_index("c")
    s = lax.axis_index("s")
    @pl.when(c == 0)
    def _():
      pltpu.sync_copy(zr, cacc.at[pl.ds(s * STRIPE, STRIPE)])
      pltpu.sync_copy(dstidx.at[s], dstv)
      pltpu.sync_copy(ones_h, onesv)
    plsc.subcore_barrier()
    @pl.when(c == 0)
    def _():
      def chunk(j, carry):
        pltpu.sync_copy(onesv, cacc.at[dstv.at[j]], add=True)
        return carry
      lax.fori_loop(0, NCHUNK, chunk, 0)
    plsc.subcore_barrier()
    @pl.when(c == 0)
    def _():
      pltpu.sync_copy(cacc.at[pl.ds(s * STRIPE, STRIPE)],
                      cntout.at[pl.ds(s * STRIPE, STRIPE)])

  return pl.kernel(
      body,
      out_type=jax.ShapeDtypeStruct((NPAD, CH), jnp.float32),
      mesh=mesh,
      scratch_types=[
          pltpu.VMEM((NCHUNK, C), jnp.int32),
          pltpu.VMEM((C, CH), jnp.float32),
          pltpu.VMEM_SHARED((NPAD, CH), jnp.float32),
      ])


def _mm_body(agg_ref, cnt_ref, h_ref, wl_ref, bl_ref, wr_ref, o_ref):
  a = agg_ref[...] / jnp.maximum(cnt_ref[...], 1.0)
  o = (jnp.dot(a, wl_ref[...], preferred_element_type=jnp.float32)
       + bl_ref[...]
       + jnp.dot(h_ref[...], wr_ref[...], preferred_element_type=jnp.float32))
  o_ref[...] = jnp.maximum(o, 0.0)


def _sage_dense(agg, cnt2, h, wl, bl2, wr):
  return pl.pallas_call(
      _mm_body,
      grid=(N // RB,),
      in_specs=[
          pl.BlockSpec((RB, H), lambda i: (i, 0)),
          pl.BlockSpec((RB, 1), lambda i: (i, 0)),
          pl.BlockSpec((RB, H), lambda i: (i, 0)),
          pl.BlockSpec((H, H), lambda i: (0, 0)),
          pl.BlockSpec((1, H), lambda i: (0, 0)),
          pl.BlockSpec((H, H), lambda i: (0, 0)),
      ],
      out_specs=pl.BlockSpec((RB, H), lambda i: (i, 0)),
      out_shape=jax.ShapeDtypeStruct((N, H), jnp.float32),
  )(agg, cnt2, h, wl, bl2, wr)


GB = 8


def _pool_idx_body(keys_ref, bt_ref, oi_ref):
  g0 = pl.program_id(0) * GB
  neg = jnp.float32(-jnp.inf)
  big = jnp.int32(2 ** 30)
  gvec = g0 + lax.broadcasted_iota(jnp.int32, (GB, 1), 0)
  flat = lax.broadcasted_iota(jnp.int32, (GB, NP2), 1)
  masked = jnp.where(bt_ref[...] == gvec, keys_ref[...], neg)
  for k in range(K):
    m = jnp.max(masked, axis=1, keepdims=True)
    idx = jnp.min(jnp.where(masked == m, flat, big), axis=1, keepdims=True)
    oi_ref[:, k:k + 1] = jnp.where(m > neg, idx, N)
    masked = jnp.where(flat == idx, neg, masked)


def _pool_idx(keys, bt):
  return pl.pallas_call(
      _pool_idx_body,
      grid=(B // GB,),
      in_specs=[
          pl.BlockSpec((1, NP2), lambda i: (0, 0)),
          pl.BlockSpec((1, NP2), lambda i: (0, 0)),
      ],
      out_specs=pl.BlockSpec((GB, K), lambda i: (i, 0)),
      out_shape=jax.ShapeDtypeStruct((B, K), jnp.int32),
  )(keys, bt)


GROWS = B * K // (NC * NS)
GPAD = 64


@functools.cache
def _make_rowgather():
  mesh = plsc.VectorSubcoreMesh(
      core_axis_name="c", subcore_axis_name="s", num_cores=NC, num_subcores=NS)

  def body(hpad, idx, out, idxv, rowsv):
    c = lax.axis_index("c")
    s = lax.axis_index("s")
    w = c * NS + s
    pltpu.sync_copy(idx.at[w], idxv)
    pltpu.sync_copy(hpad.at[idxv], rowsv)
    pltpu.sync_copy(rowsv, out.at[w])

  return pl.kernel(
      body,
      out_type=jax.ShapeDtypeStruct((NC * NS, GPAD, H), jnp.float32),
      mesh=mesh,
      scratch_types=[
          pltpu.VMEM((GPAD,), jnp.int32),
          pltpu.VMEM((GPAD, H), jnp.float32),
      ])


def _split(h):
  return jnp.concatenate([h[:, :CH], h[:, CH:]], axis=0)


def _unsplit(aggout):
  return jnp.concatenate([aggout[0, :N], aggout[1, :N]], axis=1)


@jax.jit
def kernel(x, edge_index, batch, Wl1, bl1, Wr1, Wl2, bl2, Wr2, Wl3, bl3, Wr3):
  src = edge_index[0].astype(jnp.int32)
  dst = edge_index[1].astype(jnp.int32)
  src_t = src.reshape(NS, NCHUNK, C)
  srcidx = jnp.stack([src_t, src_t + N])
  dstidx = dst.reshape(NS, NCHUNK, C)
  zr = jnp.zeros((STRIPE, CH), jnp.float32)
  ones_h = jnp.ones((C, CH), jnp.float32)

  cntout = _make_cnt()(dstidx, zr, ones_h)
  cnt2 = cntout[:N, :1]

  h = x
  for wl, bl, wr in ((Wl1, bl1, Wr1), (Wl2, bl2, Wr2), (Wl3, bl3, Wr3)):
    hs = _split(h)
    aggout = _make_agg()(hs, srcidx, dstidx, zr)
    agg = _unsplit(aggout)
    h = _sage_dense(agg, cnt2, h, wl, bl.reshape(1, H), wr)

  keys = jnp.concatenate(
      [h[:, -1], jnp.full((NP2 - N,), -jnp.inf, jnp.float32)]
  ).reshape(1, NP2)
  bt = jnp.concatenate(
      [batch.astype(jnp.int32), jnp.full((NP2 - N,), -1, jnp.int32)]
  ).reshape(1, NP2)
  topidx = _pool_idx(keys, bt)
  hpad = jnp.concatenate([h, jnp.zeros((16, H), jnp.float32)], axis=0)
  idx64 = jnp.full((NC * NS, GPAD), N, jnp.int32)
  idx64 = idx64.at[:, :GROWS].set(topidx.reshape(NC * NS, GROWS))
  rows = _make_rowgather()(hpad, idx64)
  return rows[:, :GROWS].reshape(B, K * H)
